# Initial kernel scaffold; baseline (speedup 1.0000x reference)
import jax
import jax.numpy as jnp
from jax.experimental import pallas as pl


def kernel(input_ids, attention_mask, table):
    raise NotImplementedError("write your pallas kernel here")


# speedup vs baseline: 1.1775x; 1.1775x over previous
import functools

import jax
import jax.numpy as jnp
from jax import lax
from jax.experimental import pallas as pl
from jax.experimental.pallas import tpu as pltpu
from jax.experimental.pallas import tpu_sc as plsc

_VOCAB = 50257
_DIM = 768
_BATCH = 64
_SEQ = 1024
_TOK = _BATCH * _SEQ

_NC = 2
_NS = 16
_NW = _NC * _NS
_TPW = _TOK // _NW
_CH = 32
_NCH = _TPW // _CH
_LANES = 16
_DREGS = _DIM // _LANES


def _body(ids_hbm, mask_hbm, table_hbm, out_hbm,
          idx_v, mask_v, in_v, out_v, gsem0, gsem1, osem0, osem1):
    wid = lax.axis_index("c") * _NS + lax.axis_index("s")
    base = wid * _TPW
    gsems = (gsem0, gsem1)
    osems = (osem0, osem1)

    pltpu.sync_copy(ids_hbm.at[wid], idx_v)
    pltpu.sync_copy(mask_hbm.at[wid], mask_v)

    def start_gather(i, b):
        pltpu.async_copy(table_hbm.at[idx_v.at[i]], in_v.at[b], gsems[b])

    def out_dst(i):
        return out_hbm.at[pl.ds(base + i * _CH, _CH)]

    start_gather(0, 0)
    start_gather(1, 1)

    def chunk(i, b):
        pltpu.make_async_copy(table_hbm.at[idx_v.at[i]], in_v.at[b],
                              gsems[b]).wait()
        @pl.when(i >= 2)
        def _():
            pltpu.make_async_copy(out_v.at[b], out_dst(i), osems[b]).wait()

        def group(g, _):
            gbase = g * _LANES
            m16 = mask_v[pl.ds(i * _CH + gbase, _LANES)]
            for t in range(_LANES):
                m = m16[t]
                for j in range(_DREGS):
                    sl = pl.ds(j * _LANES, _LANES)
                    out_v[b, gbase + t, sl] = in_v[b, gbase + t, sl] * m
            return 0

        lax.fori_loop(0, _CH // _LANES, group, 0, unroll=False)

        pltpu.async_copy(out_v.at[b], out_dst(i), osems[b])

        @pl.when(i + 2 < _NCH)
        def _():
            start_gather(i + 2, b)

    def pair(io, _):
        chunk(io, 0)
        chunk(io + 1, 1)
        return 0

    lax.fori_loop(0, _NCH // 2, lambda k, c: pair(k * 2, c), 0, unroll=False)

    pltpu.make_async_copy(out_v.at[0], out_dst(_NCH - 2), osem0).wait()
    pltpu.make_async_copy(out_v.at[1], out_dst(_NCH - 1), osem1).wait()


@jax.jit
def _lookup(ids, mask_f, table):
    mesh = plsc.VectorSubcoreMesh(core_axis_name="c", subcore_axis_name="s")
    run = pl.kernel(
        _body,
        out_type=jax.ShapeDtypeStruct((_TOK, _DIM), jnp.float32),
        mesh=mesh,
        scratch_types=[
            pltpu.VMEM((_NCH, _CH), jnp.int32),
            pltpu.VMEM((_TPW,), jnp.float32),
            pltpu.VMEM((2, _CH, _DIM), jnp.float32),
            pltpu.VMEM((2, _CH, _DIM), jnp.float32),
            pltpu.SemaphoreType.DMA,
            pltpu.SemaphoreType.DMA,
            pltpu.SemaphoreType.DMA,
            pltpu.SemaphoreType.DMA,
        ],
    )
    return run(ids, mask_f, table)


def kernel(input_ids, attention_mask, table):
    ids = input_ids.reshape(_NW, _NCH, _CH).astype(jnp.int32)
    mask_f = attention_mask.reshape(_NW, _TPW).astype(jnp.float32)
    out = _lookup(ids, mask_f, table)
    return out.reshape(_BATCH, _SEQ, _DIM), attention_mask
